# baseline (speedup 1.0000x reference)
import jax
import jax.numpy as jnp
from jax import lax
from jax.experimental import pallas as pl
from jax.experimental.pallas import tpu as pltpu
from jax.experimental.pallas import tpu_sc as plsc

N_I = 16384
N_J = 200
D = 32
NW = 32
STRIP = N_I // NW
LB = STRIP // 128


def _sc_body(xt3, table, out5,
             idx0, idx1, rows0, rows1, tile0, tile1,
             si0, si1, sg0, sg1, so0, so1):
    wid = lax.axis_index("s") * 2 + lax.axis_index("c")
    ihi0 = wid * LB

    bufs = ((idx0, rows0, tile0, si0, sg0, so0),
            (idx1, rows1, tile1, si1, sg1, so1))
    lane = jnp.arange(16, dtype=jnp.int32)

    def idx_start(j, idxv, sem):
        pltpu.async_copy(xt3.at[j, pl.ds(ihi0, LB)], idxv, sem)

    def gather_start(idxv, rowsv, sem):
        for lb in range(LB):
            pltpu.async_copy(
                table.at[idxv.at[lb]],
                rowsv.at[pl.ds(lb * 128, 128)],
                sem,
            )

    def transpose_block(lb, rowsv, tilev):
        def body(r, carry):
            for tr in range(4):
                col = jnp.full((16,), tr * 8 + r, dtype=jnp.int32)
                for lg in range(8):
                    row = lane + (lb * 128 + lg * 16)
                    v = plsc.load_gather(rowsv, [row, col])
                    tilev[tr, lb, r, pl.ds(lg * 16, 16)] = v
            return carry
        lax.fori_loop(0, 8, body, 0)

    def out_start(j, tilev, sem):
        for tr in range(4):
            pltpu.async_copy(
                tilev.at[tr],
                out5.at[j, tr, pl.ds(ihi0, LB)],
                sem,
            )

    def out_wait(tilev, sem):
        for tr in range(4):
            pltpu.make_async_copy(tilev.at[tr], out5.at[0, 0, pl.ds(0, LB)], sem).wait()

    idx_start(0, idx0, si0)
    pltpu.make_async_copy(xt3.at[0, pl.ds(0, LB)], idx0, si0).wait()
    gather_start(idx0, rows0, sg0)
    idx_start(1, idx1, si1)

    def pair(p, carry):
        for b in range(2):
            idxv, rowsv, tilev, si, sg, so = bufs[b]
            n_idxv, n_rowsv, _, n_si, n_sg, _ = bufs[1 - b]
            j = 2 * p + b

            @pl.when(j >= 2)
            def _():
                out_wait(tilev, so)

            for lb in range(LB):
                pltpu.make_async_copy(
                    table.at[idxv.at[lb]],
                    rowsv.at[pl.ds(lb * 128, 128)],
                    sg,
                ).wait()
                transpose_block(lb, rowsv, tilev)

            @pl.when(j + 2 < N_J)
            def _():
                idx_start(j + 2, idxv, si)

            out_start(j, tilev, so)

            @pl.when(j + 1 < N_J)
            def _():
                pltpu.make_async_copy(xt3.at[0, pl.ds(0, LB)], n_idxv, n_si).wait()
                gather_start(n_idxv, n_rowsv, n_sg)
        return carry

    lax.fori_loop(0, N_J // 2, pair, 0)

    out_wait(tile0, so0)
    out_wait(tile1, so1)


def kernel(x, table):
    xt3 = x.T.astype(jnp.int32).reshape(N_J, N_I // 128, 128)
    mesh = plsc.VectorSubcoreMesh(core_axis_name="c", subcore_axis_name="s")
    out5 = pl.kernel(
        _sc_body,
        out_type=jax.ShapeDtypeStruct((N_J, 4, N_I // 128, 8, 128), jnp.float32),
        mesh=mesh,
        scratch_types=[
            pltpu.VMEM((LB, 128), jnp.int32),
            pltpu.VMEM((LB, 128), jnp.int32),
            pltpu.VMEM((STRIP, D), jnp.float32),
            pltpu.VMEM((STRIP, D), jnp.float32),
            pltpu.VMEM((4, LB, 8, 128), jnp.float32),
            pltpu.VMEM((4, LB, 8, 128), jnp.float32),
            pltpu.SemaphoreType.DMA,
            pltpu.SemaphoreType.DMA,
            pltpu.SemaphoreType.DMA,
            pltpu.SemaphoreType.DMA,
            pltpu.SemaphoreType.DMA,
            pltpu.SemaphoreType.DMA,
        ],
        compiler_params=pltpu.CompilerParams(
            use_tc_tiling_on_sc=False, needs_layout_passes=False
        ),
    )(xt3, table)
    return out5.transpose(2, 4, 0, 1, 3).reshape(N_I, N_J, D)


# speedup vs baseline: 3.6940x; 3.6940x over previous
import jax
import jax.numpy as jnp
from jax import lax
from jax.experimental import pallas as pl
from jax.experimental.pallas import tpu as pltpu
from jax.experimental.pallas import tpu_sc as plsc

N_I = 16384
N_J = 200
D = 32
NW = 32
STRIP = N_I // NW
LB = STRIP // 128
PITCH = 513


def _sc_body(xt3, table, out5,
             idx0, idx1, rows0, rows1, tile0, tile1,
             si0, si1, sg0, sg1, so0, so1):
    wid = lax.axis_index("s") * 2 + lax.axis_index("c")
    ihi0 = wid * LB

    bufs = ((idx0, rows0, tile0, si0, sg0, so0),
            (idx1, rows1, tile1, si1, sg1, so1))
    lane = jnp.arange(16, dtype=jnp.int32)

    def idx_start(j, idxv, sem):
        pltpu.async_copy(xt3.at[j, pl.ds(ihi0, LB)], idxv, sem)

    def gather_start(idxv, rowsv, sem):
        for lb in range(LB):
            pltpu.async_copy(
                table.at[idxv.at[lb]],
                rowsv.at[pl.ds(lb * 128, 128)],
                sem,
            )

    def transpose_block(lb, rowsv, tilev):
        rows_lo = lane
        rows_hi = lane + 16

        def body(k, carry):
            rv = lb * 128 + k * 8
            vs = [
                (rowsv[rv + u, pl.ds(0, 16)], rowsv[rv + u, pl.ds(16, 16)])
                for u in range(8)
            ]
            for u in range(8):
                col = jnp.full((16,), rv + u, dtype=jnp.int32)
                plsc.store_scatter(tilev, [rows_lo, col], vs[u][0])
                plsc.store_scatter(tilev, [rows_hi, col], vs[u][1])
            return carry
        lax.fori_loop(0, 16, body, 0)

    def out_start(j, tilev, sem):
        for tr in range(4):
            for lb in range(LB):
                pltpu.async_copy(
                    tilev.at[pl.ds(tr * 8, 8), pl.ds(lb * 128, 128)],
                    out5.at[j, tr, ihi0 + lb],
                    sem,
                )

    def out_wait(tilev, sem):
        for _ in range(4 * LB):
            pltpu.make_async_copy(
                tilev.at[pl.ds(0, 8), pl.ds(0, 128)],
                out5.at[0, 0, 0],
                sem,
            ).wait()

    idx_start(0, idx0, si0)
    pltpu.make_async_copy(xt3.at[0, pl.ds(0, LB)], idx0, si0).wait()
    gather_start(idx0, rows0, sg0)
    idx_start(1, idx1, si1)

    def pair(p, carry):
        for b in range(2):
            idxv, rowsv, tilev, si, sg, so = bufs[b]
            n_idxv, n_rowsv, _, n_si, n_sg, _ = bufs[1 - b]
            j = 2 * p + b

            @pl.when(j >= 2)
            def _():
                out_wait(tilev, so)

            for lb in range(LB):
                pltpu.make_async_copy(
                    table.at[idxv.at[lb]],
                    rowsv.at[pl.ds(lb * 128, 128)],
                    sg,
                ).wait()
                transpose_block(lb, rowsv, tilev)

            @pl.when(j + 2 < N_J)
            def _():
                idx_start(j + 2, idxv, si)

            out_start(j, tilev, so)

            @pl.when(j + 1 < N_J)
            def _():
                pltpu.make_async_copy(xt3.at[0, pl.ds(0, LB)], n_idxv, n_si).wait()
                gather_start(n_idxv, n_rowsv, n_sg)
        return carry

    lax.fori_loop(0, N_J // 2, pair, 0)

    out_wait(tile0, so0)
    out_wait(tile1, so1)


def kernel(x, table):
    xt3 = x.T.astype(jnp.int32).reshape(N_J, N_I // 128, 128)
    mesh = plsc.VectorSubcoreMesh(core_axis_name="c", subcore_axis_name="s")
    out5 = pl.kernel(
        _sc_body,
        out_type=jax.ShapeDtypeStruct((N_J, 4, N_I // 128, 8, 128), jnp.float32),
        mesh=mesh,
        scratch_types=[
            pltpu.VMEM((LB, 128), jnp.int32),
            pltpu.VMEM((LB, 128), jnp.int32),
            pltpu.VMEM((STRIP, D), jnp.float32),
            pltpu.VMEM((STRIP, D), jnp.float32),
            pltpu.VMEM((D, PITCH), jnp.float32),
            pltpu.VMEM((D, PITCH), jnp.float32),
            pltpu.SemaphoreType.DMA,
            pltpu.SemaphoreType.DMA,
            pltpu.SemaphoreType.DMA,
            pltpu.SemaphoreType.DMA,
            pltpu.SemaphoreType.DMA,
            pltpu.SemaphoreType.DMA,
        ],
        compiler_params=pltpu.CompilerParams(
            use_tc_tiling_on_sc=False, needs_layout_passes=False
        ),
    )(xt3, table)
    return out5.transpose(2, 4, 0, 1, 3).reshape(N_I, N_J, D)
